# Initial kernel scaffold; baseline (speedup 1.0000x reference)
import jax
import jax.numpy as jnp
from jax.experimental import pallas as pl


def kernel(x, edge_index, edge_attr, p_spec, q_spec, node_gs, node_bs, bus_type, vm_sp):
    raise NotImplementedError("write your pallas kernel here")


# speedup vs baseline: 245.8002x; 245.8002x over previous
import functools

import jax
import jax.numpy as jnp
from jax import lax
from jax.experimental import pallas as pl
from jax.experimental.pallas import tpu as pltpu
from jax.experimental.pallas import tpu_sc as plsc

_STEP = 0.1
_VM_MIN, _VM_MAX = 0.9, 1.1
_N_ITERS = 2
_EPS = 1e-12

_NSC = 2
_NTILE = 16
_NW = _NSC * _NTILE
_L = 16

_S = (0.999999599900364, -0.1666655263107888, 0.008332402961170623,
      -0.0001980863262521467, 2.699713829178163e-06, -2.0362212166391558e-08)
_C = (0.9999992107412048, -0.49999421314963205, 0.041659777585706076,
      -0.0013858789204440978, 2.4202932052880266e-05, -2.1972921876445284e-07)
_INV2PI = 0.15915494309189535
_MAGIC = 12582912.0
_P2_HI = 6.283185482025146
_P2_LO = -1.7484556000744883e-07


def _sincos(d):
    nf = (d * _INV2PI + _MAGIC) - _MAGIC
    r = d - nf * _P2_HI
    r = r - nf * _P2_LO
    u = r * r
    sp = u * _S[5] + _S[4]
    sp = u * sp + _S[3]
    sp = u * sp + _S[2]
    sp = u * sp + _S[1]
    sp = u * sp + _S[0]
    cp = u * _C[5] + _C[4]
    cp = u * cp + _C[3]
    cp = u * cp + _C[2]
    cp = u * cp + _C[1]
    cp = u * cp + _C[0]
    return r * sp, cp


def _precompute_body(r_ref, x_ref, gfr_ref, bfr_ref, gto_ref, bto_ref,
                     tau_ref, sh_ref, out_ref):
    r = r_ref[...]
    xx = x_ref[...]
    denom = r * r + xx * xx + _EPS
    g_s = r / denom
    b_s = -xx / denom
    it = 1.0 / tau_ref[...]
    sh = sh_ref[...]
    cs = jnp.cos(sh)
    ss = jnp.sin(sh)
    out_ref[0] = (g_s * cs - b_s * ss) * it
    out_ref[1] = (g_s * ss + b_s * cs) * it
    out_ref[2] = (g_s * cs + b_s * ss) * it
    out_ref[3] = (b_s * cs - g_s * ss) * it
    it2 = it * it
    out_ref[4] = (g_s + gfr_ref[...]) * it2
    out_ref[5] = (b_s + bfr_ref[...]) * it2
    out_ref[6] = g_s + gto_ref[...]
    out_ref[7] = b_s + bto_ref[...]


def _precompute_econ(cols, n_rows, blk):
    grid = n_rows // blk
    in_spec = pl.BlockSpec((blk, 128), lambda i: (i, 0))
    out_spec = pl.BlockSpec((8, blk, 128), lambda i: (0, i, 0))
    return pl.pallas_call(
        _precompute_body,
        grid=(grid,),
        in_specs=[in_spec] * 8,
        out_specs=out_spec,
        out_shape=jax.ShapeDtypeStruct((8, n_rows, 128), jnp.float32),
    )(*cols)


def _edge_body(n_pad, ept, chunk, e_total,
               va_hbm, vm_hbm, src_hbm, dst_hbm, econ_hbm,
               p_out, q_out,
               va_sh, vm_sh, p_sh, q_sh,
               srcv, dstv, thiv, thjv, viv, vjv,
               ec0, ec1, ec2, ec3, ec4, ec5, ec6, ec7,
               pfv, qfv, ptv, qtv, zrow,
               sem_in, sem_g, sem_s):
    econ_v = (ec0, ec1, ec2, ec3, ec4, ec5, ec6, ec7)
    c = lax.axis_index("c")
    s = lax.axis_index("s")
    wid = c * _NTILE + s
    nslice = n_pad // _NTILE
    base = s * nslice

    def _zero_step(i, carry):
        zrow[pl.ds(i * _L, _L)] = jnp.zeros((_L,), jnp.float32)
        return carry

    lax.fori_loop(0, nslice // _L, _zero_step, 0)
    pltpu.sync_copy(zrow, p_sh.at[pl.ds(base, nslice)])
    pltpu.sync_copy(zrow, q_sh.at[pl.ds(base, nslice)])
    pltpu.sync_copy(va_hbm.at[pl.ds(base, nslice)], va_sh.at[pl.ds(base, nslice)])
    pltpu.sync_copy(vm_hbm.at[pl.ds(base, nslice)], vm_sh.at[pl.ds(base, nslice)])
    plsc.subcore_barrier()

    ebase = wid * ept

    def _chunk(k, carry):
        eoff = ebase + k * chunk
        cps = [pltpu.async_copy(src_hbm.at[pl.ds(eoff, chunk)], srcv, sem_in),
               pltpu.async_copy(dst_hbm.at[pl.ds(eoff, chunk)], dstv, sem_in)]
        for g in range(8):
            cps.append(pltpu.async_copy(
                econ_hbm.at[pl.ds(g * e_total + eoff, chunk)],
                econ_v[g], sem_in))
        for cp in cps:
            cp.wait()
        gs = [pltpu.async_copy(va_sh.at[srcv], thiv, sem_g),
              pltpu.async_copy(va_sh.at[dstv], thjv, sem_g),
              pltpu.async_copy(vm_sh.at[srcv], viv, sem_g),
              pltpu.async_copy(vm_sh.at[dstv], vjv, sem_g)]
        for cp in gs:
            cp.wait()

        def _cstep(i, carry2):
            sl = pl.ds(i * _L, _L)
            thi = thiv[sl]
            thj = thjv[sl]
            vi = viv[sl]
            vj = vjv[sl]
            sd, cd = _sincos(thi - thj)
            grt = ec0[sl]
            brt = ec1[sl]
            gr2 = ec2[sl]
            br2 = ec3[sl]
            vi2 = vi * vi
            vj2 = vj * vj
            vij = vi * vj
            taf = grt * cd + brt * sd
            tbf = grt * sd - brt * cd
            tat = gr2 * cd - br2 * sd
            tbt = br2 * cd + gr2 * sd
            pfv[sl] = vi2 * ec4[sl] - vij * taf
            qfv[sl] = -(vi2 * ec5[sl] + vij * tbf)
            ptv[sl] = vj2 * ec6[sl] - vij * tat
            qtv[sl] = -(vj2 * ec7[sl] - vij * tbt)
            return carry2

        lax.fori_loop(0, chunk // _L, _cstep, 0)
        ss_ = [pltpu.async_copy(pfv, p_sh.at[srcv], sem_s, add=True),
               pltpu.async_copy(qfv, q_sh.at[srcv], sem_s, add=True),
               pltpu.async_copy(ptv, p_sh.at[dstv], sem_s, add=True),
               pltpu.async_copy(qtv, q_sh.at[dstv], sem_s, add=True)]
        for cp in ss_:
            cp.wait()
        return carry

    lax.fori_loop(0, ept // chunk, _chunk, 0)
    plsc.subcore_barrier()
    pltpu.sync_copy(p_sh.at[pl.ds(base, nslice)], p_out.at[c, pl.ds(base, nslice)])
    pltpu.sync_copy(q_sh.at[pl.ds(base, nslice)], q_out.at[c, pl.ds(base, nslice)])


def _edge_pass(vap, vmp, src, dst, econ, n_pad, ept, chunk, e_total):
    mesh = plsc.VectorSubcoreMesh(core_axis_name="c", subcore_axis_name="s")
    body = functools.partial(_edge_body, n_pad, ept, chunk, e_total)
    f = pl.kernel(
        body,
        out_type=(jax.ShapeDtypeStruct((_NSC, n_pad), jnp.float32),
                  jax.ShapeDtypeStruct((_NSC, n_pad), jnp.float32)),
        mesh=mesh,
        scratch_types=[
            pltpu.VMEM_SHARED((n_pad,), jnp.float32),
            pltpu.VMEM_SHARED((n_pad,), jnp.float32),
            pltpu.VMEM_SHARED((n_pad,), jnp.float32),
            pltpu.VMEM_SHARED((n_pad,), jnp.float32),
            pltpu.VMEM((chunk,), jnp.int32),
            pltpu.VMEM((chunk,), jnp.int32),
            pltpu.VMEM((chunk,), jnp.float32),
            pltpu.VMEM((chunk,), jnp.float32),
            pltpu.VMEM((chunk,), jnp.float32),
            pltpu.VMEM((chunk,), jnp.float32),
            pltpu.VMEM((chunk,), jnp.float32),
            pltpu.VMEM((chunk,), jnp.float32),
            pltpu.VMEM((chunk,), jnp.float32),
            pltpu.VMEM((chunk,), jnp.float32),
            pltpu.VMEM((chunk,), jnp.float32),
            pltpu.VMEM((chunk,), jnp.float32),
            pltpu.VMEM((chunk,), jnp.float32),
            pltpu.VMEM((chunk,), jnp.float32),
            pltpu.VMEM((chunk,), jnp.float32),
            pltpu.VMEM((chunk,), jnp.float32),
            pltpu.VMEM((chunk,), jnp.float32),
            pltpu.VMEM((chunk,), jnp.float32),
            pltpu.VMEM((n_pad // _NTILE,), jnp.float32),
            pltpu.SemaphoreType.DMA,
            pltpu.SemaphoreType.DMA,
            pltpu.SemaphoreType.DMA,
        ],
    )
    return f(vap, vmp, src, dst, econ)


def _node_body(final, pp_ref, qq_ref, va_ref, vm_ref, psp_ref, qsp_ref,
               gs_ref, bs_ref, bt_ref, vsp_ref, oa_ref, ob_ref):
    va = va_ref[...]
    vm = vm_ref[...]
    vm2 = vm * vm
    p_calc = pp_ref[0] + pp_ref[1] + vm2 * gs_ref[...]
    q_calc = qq_ref[0] + qq_ref[1] - vm2 * bs_ref[...]
    bt = bt_ref[...]
    pv = bt == 2
    sl = bt == 3
    f_p = jnp.where(sl, va, psp_ref[...] - p_calc)
    f_q = jnp.where(pv | sl, vm - vsp_ref[...], qsp_ref[...] - q_calc)
    if final:
        oa_ref[...] = f_p
        ob_ref[...] = f_q
    else:
        oa_ref[...] = va - _STEP * f_p
        ob_ref[...] = jnp.clip(vm - _STEP * f_q, _VM_MIN, _VM_MAX)


def _node_pass(final, pq, va2, vm2, psp, qsp, gs, bs, bt, vsp):
    p_parts, q_parts = pq
    rows = va2.shape[0]
    p3 = p_parts.reshape(_NSC, rows, 128)
    q3 = q_parts.reshape(_NSC, rows, 128)
    out_sd = jax.ShapeDtypeStruct((rows, 128), jnp.float32)
    return pl.pallas_call(
        functools.partial(_node_body, final),
        out_shape=(out_sd, out_sd),
    )(p3, q3, va2, vm2, psp, qsp, gs, bs, bt, vsp)


def kernel(x, edge_index, edge_attr, p_spec, q_spec, node_gs, node_bs,
           bus_type, vm_sp):
    n = x.shape[1] // 2
    e_total = edge_index.shape[1]
    n_pad = ((n + 2047) // 2048) * 2048
    rows = n_pad // 128
    ept = e_total // _NW
    chunk = 4000
    assert ept % chunk == 0 and chunk % _L == 0

    pad = n_pad - n
    vap = jnp.pad(x[0, :n], (0, pad))
    vmp = jnp.pad(x[0, n:], (0, pad))
    src = edge_index[0]
    dst = edge_index[1]

    e_rows = e_total // 128
    cols = [edge_attr[:, k].reshape(e_rows, 128) for k in range(8)]
    econ = _precompute_econ(cols, e_rows, 200).reshape(8 * e_total)

    def p2(a):
        return jnp.pad(a[0], (0, pad)).reshape(rows, 128)

    psp = p2(p_spec)
    qsp = p2(q_spec)
    gs = p2(node_gs)
    bs = p2(node_bs)
    vsp = p2(vm_sp)
    bt = jnp.pad(bus_type[0], (0, pad), constant_values=1).reshape(rows, 128)

    va2 = vap.reshape(rows, 128)
    vm2 = vmp.reshape(rows, 128)
    for _ in range(_N_ITERS):
        pq = _edge_pass(va2.reshape(n_pad), vm2.reshape(n_pad), src, dst,
                        econ, n_pad, ept, chunk, e_total)
        va2, vm2 = _node_pass(False, pq, va2, vm2, psp, qsp, gs, bs, bt, vsp)
    pq = _edge_pass(va2.reshape(n_pad), vm2.reshape(n_pad), src, dst,
                    econ, n_pad, ept, chunk, e_total)
    f_p, f_q = _node_pass(True, pq, va2, vm2, psp, qsp, gs, bs, bt, vsp)
    return jnp.concatenate([f_p.reshape(n_pad)[:n][None, :],
                            f_q.reshape(n_pad)[:n][None, :]], axis=1)


# speedup vs baseline: 264.6414x; 1.0767x over previous
import functools

import jax
import jax.numpy as jnp
from jax import lax
from jax.experimental import pallas as pl
from jax.experimental.pallas import tpu as pltpu
from jax.experimental.pallas import tpu_sc as plsc

_STEP = 0.1
_VM_MIN, _VM_MAX = 0.9, 1.1
_N_ITERS = 2
_EPS = 1e-12

_NSC = 2
_NTILE = 16
_NW = _NSC * _NTILE
_L = 16

_S = (0.999999599900364, -0.1666655263107888, 0.008332402961170623,
      -0.0001980863262521467, 2.699713829178163e-06, -2.0362212166391558e-08)
_C = (0.9999992107412048, -0.49999421314963205, 0.041659777585706076,
      -0.0013858789204440978, 2.4202932052880266e-05, -2.1972921876445284e-07)
_INV2PI = 0.15915494309189535
_MAGIC = 12582912.0
_P2_HI = 6.283185482025146
_P2_LO = -1.7484556000744883e-07


def _sincos(d):
    nf = (d * _INV2PI + _MAGIC) - _MAGIC
    r = d - nf * _P2_HI
    r = r - nf * _P2_LO
    u = r * r
    sp = u * _S[5] + _S[4]
    sp = u * sp + _S[3]
    sp = u * sp + _S[2]
    sp = u * sp + _S[1]
    sp = u * sp + _S[0]
    cp = u * _C[5] + _C[4]
    cp = u * cp + _C[3]
    cp = u * cp + _C[2]
    cp = u * cp + _C[1]
    cp = u * cp + _C[0]
    return r * sp, cp


def _precompute_body(att_ref, out_ref):
    r = att_ref[:, 0, :]
    xx = att_ref[:, 1, :]
    denom = r * r + xx * xx + _EPS
    g_s = r / denom
    b_s = -xx / denom
    it = 1.0 / att_ref[:, 6, :]
    sh = att_ref[:, 7, :]
    cs = jnp.cos(sh)
    ss = jnp.sin(sh)
    out_ref[0] = (g_s * cs - b_s * ss) * it
    out_ref[1] = (g_s * ss + b_s * cs) * it
    out_ref[2] = (g_s * cs + b_s * ss) * it
    out_ref[3] = (b_s * cs - g_s * ss) * it
    it2 = it * it
    out_ref[4] = (g_s + att_ref[:, 2, :]) * it2
    out_ref[5] = (b_s + att_ref[:, 3, :]) * it2
    out_ref[6] = g_s + att_ref[:, 4, :]
    out_ref[7] = b_s + att_ref[:, 5, :]


def _precompute_econ(att, n_rows, blk):
    grid = n_rows // blk
    in_spec = pl.BlockSpec((blk, 8, 128), lambda i: (i, 0, 0))
    out_spec = pl.BlockSpec((8, blk, 128), lambda i: (0, i, 0))
    return pl.pallas_call(
        _precompute_body,
        grid=(grid,),
        in_specs=[in_spec],
        out_specs=out_spec,
        out_shape=jax.ShapeDtypeStruct((8, n_rows, 128), jnp.float32),
    )(att)


def _edge_body(n_pad, ept, chunk, e_total,
               va_hbm, vm_hbm, src_hbm, dst_hbm, econ_hbm,
               p_out, q_out,
               va_sh, vm_sh, p_sh, q_sh,
               *scr):
    data = (scr[0:16], scr[16:32])
    idx = (scr[32:34], scr[34:36], scr[36:38], scr[38:40])
    zrow = scr[40]
    sem_in, sem_g = scr[41], scr[42]
    sem_s = (scr[43], scr[44])
    c = lax.axis_index("c")
    s = lax.axis_index("s")
    wid = c * _NTILE + s
    nslice = n_pad // _NTILE
    base = s * nslice
    nchunks = ept // chunk
    cpad = chunk + (-chunk) % _L
    ebase = wid * ept

    def _zero_step(i, carry):
        zrow[pl.ds(i * _L, _L)] = jnp.zeros((_L,), jnp.float32)
        return carry

    lax.fori_loop(0, nslice // _L, _zero_step, 0)
    pltpu.sync_copy(zrow, p_sh.at[pl.ds(base, nslice)])
    pltpu.sync_copy(zrow, q_sh.at[pl.ds(base, nslice)])
    pltpu.sync_copy(va_hbm.at[pl.ds(base, nslice)], va_sh.at[pl.ds(base, nslice)])
    pltpu.sync_copy(vm_hbm.at[pl.ds(base, nslice)], vm_sh.at[pl.ds(base, nslice)])
    plsc.subcore_barrier()

    def _lin_issue(b, k):
        eoff = ebase + k * chunk
        D = data[b % 2]
        ix = idx[b % 4]
        pltpu.async_copy(src_hbm.at[pl.ds(eoff, chunk)], ix[0], sem_in)
        pltpu.async_copy(dst_hbm.at[pl.ds(eoff, chunk)], ix[1], sem_in)
        for g in range(8):
            pltpu.async_copy(
                econ_hbm.at[pl.ds(g * e_total + eoff, chunk)],
                D[4 + g].at[pl.ds(0, chunk)], sem_in)

    def _lin_wait(b):
        D = data[b % 2]
        ix = idx[b % 4]
        pltpu.make_async_copy(src_hbm.at[pl.ds(0, chunk)], ix[0], sem_in).wait()
        pltpu.make_async_copy(dst_hbm.at[pl.ds(0, chunk)], ix[1], sem_in).wait()
        for g in range(8):
            pltpu.make_async_copy(
                econ_hbm.at[pl.ds(0, chunk)],
                D[4 + g].at[pl.ds(0, chunk)], sem_in).wait()

    def _gath_issue(b):
        D = data[b % 2]
        ix = idx[b % 4]
        pltpu.async_copy(va_sh.at[ix[0]], D[0].at[pl.ds(0, chunk)], sem_g)
        pltpu.async_copy(va_sh.at[ix[1]], D[1].at[pl.ds(0, chunk)], sem_g)
        pltpu.async_copy(vm_sh.at[ix[0]], D[2].at[pl.ds(0, chunk)], sem_g)
        pltpu.async_copy(vm_sh.at[ix[1]], D[3].at[pl.ds(0, chunk)], sem_g)

    def _gath_wait(b):
        D = data[b % 2]
        ix = idx[b % 4]
        pltpu.make_async_copy(va_sh.at[ix[0]], D[0].at[pl.ds(0, chunk)],
                              sem_g).wait()
        pltpu.make_async_copy(va_sh.at[ix[1]], D[1].at[pl.ds(0, chunk)],
                              sem_g).wait()
        pltpu.make_async_copy(vm_sh.at[ix[0]], D[2].at[pl.ds(0, chunk)],
                              sem_g).wait()
        pltpu.make_async_copy(vm_sh.at[ix[1]], D[3].at[pl.ds(0, chunk)],
                              sem_g).wait()

    def _scat_issue(b):
        D = data[b % 2]
        ix = idx[b % 4]
        pltpu.async_copy(D[12].at[pl.ds(0, chunk)], p_sh.at[ix[0]],
                         sem_s[b % 2], add=True)
        pltpu.async_copy(D[13].at[pl.ds(0, chunk)], q_sh.at[ix[0]],
                         sem_s[b % 2], add=True)
        pltpu.async_copy(D[14].at[pl.ds(0, chunk)], p_sh.at[ix[1]],
                         sem_s[b % 2], add=True)
        pltpu.async_copy(D[15].at[pl.ds(0, chunk)], q_sh.at[ix[1]],
                         sem_s[b % 2], add=True)

    def _scat_drain(b):
        D = data[b % 2]
        ix = idx[b % 4]
        pltpu.make_async_copy(D[12].at[pl.ds(0, chunk)], p_sh.at[ix[0]],
                              sem_s[b % 2]).wait()
        pltpu.make_async_copy(D[13].at[pl.ds(0, chunk)], q_sh.at[ix[0]],
                              sem_s[b % 2]).wait()
        pltpu.make_async_copy(D[14].at[pl.ds(0, chunk)], p_sh.at[ix[1]],
                              sem_s[b % 2]).wait()
        pltpu.make_async_copy(D[15].at[pl.ds(0, chunk)], q_sh.at[ix[1]],
                              sem_s[b % 2]).wait()

    def _compute(b):
        D = data[b % 2]

        def _cstep(i, carry2):
            sl = pl.ds(i * _L, _L)
            thi = D[0][sl]
            thj = D[1][sl]
            vi = D[2][sl]
            vj = D[3][sl]
            sd, cd = _sincos(thi - thj)
            grt = D[4][sl]
            brt = D[5][sl]
            gr2 = D[6][sl]
            br2 = D[7][sl]
            vi2 = vi * vi
            vj2 = vj * vj
            vij = vi * vj
            taf = grt * cd + brt * sd
            tbf = grt * sd - brt * cd
            tat = gr2 * cd - br2 * sd
            tbt = br2 * cd + gr2 * sd
            D[12][sl] = vi2 * D[8][sl] - vij * taf
            D[13][sl] = -(vi2 * D[9][sl] + vij * tbf)
            D[14][sl] = vj2 * D[10][sl] - vij * tat
            D[15][sl] = -(vj2 * D[11][sl] - vij * tbt)
            return carry2

        lax.fori_loop(0, cpad // _L, _cstep, 0)

    _lin_issue(0, 0)
    _lin_issue(1, 1)
    _lin_wait(0)
    _gath_issue(0)

    def _k4_step(k4, carry):
        for b in range(4):
            k = k4 * 4 + b

            @pl.when(k < nchunks - 1)
            def _():
                _lin_wait(b + 1)
                _gath_issue(b + 1)

            @pl.when(k >= 2)
            def _():
                _scat_drain(b + 2)

            _gath_wait(b)
            _compute(b)
            _scat_issue(b)

            @pl.when(k < nchunks - 2)
            def _():
                _lin_issue(b + 2, k + 2)

        return carry

    lax.fori_loop(0, nchunks // 4, _k4_step, 0)
    _scat_drain(2)
    _scat_drain(3)
    plsc.subcore_barrier()
    pltpu.sync_copy(p_sh.at[pl.ds(base, nslice)], p_out.at[c, pl.ds(base, nslice)])
    pltpu.sync_copy(q_sh.at[pl.ds(base, nslice)], q_out.at[c, pl.ds(base, nslice)])


def _edge_pass(vap, vmp, src, dst, econ, n_pad, ept, chunk, e_total):
    mesh = plsc.VectorSubcoreMesh(core_axis_name="c", subcore_axis_name="s")
    body = functools.partial(_edge_body, n_pad, ept, chunk, e_total)
    f = pl.kernel(
        body,
        out_type=(jax.ShapeDtypeStruct((_NSC, n_pad), jnp.float32),
                  jax.ShapeDtypeStruct((_NSC, n_pad), jnp.float32)),
        mesh=mesh,
        scratch_types=(
            [pltpu.VMEM_SHARED((n_pad,), jnp.float32)] * 4
            + [pltpu.VMEM((chunk + (-chunk) % _L,), jnp.float32)] * 32
            + [pltpu.VMEM((chunk,), jnp.int32)] * 8
            + [pltpu.VMEM((n_pad // _NTILE,), jnp.float32)]
            + [pltpu.SemaphoreType.DMA] * 4
        ),
    )
    return f(vap, vmp, src, dst, econ)


def _node_body(final, pp_ref, qq_ref, va_ref, vm_ref, psp_ref, qsp_ref,
               gs_ref, bs_ref, bt_ref, vsp_ref, oa_ref, ob_ref):
    va = va_ref[...]
    vm = vm_ref[...]
    vm2 = vm * vm
    p_calc = pp_ref[0] + pp_ref[1] + vm2 * gs_ref[...]
    q_calc = qq_ref[0] + qq_ref[1] - vm2 * bs_ref[...]
    bt = bt_ref[...]
    pv = bt == 2
    sl = bt == 3
    f_p = jnp.where(sl, va, psp_ref[...] - p_calc)
    f_q = jnp.where(pv | sl, vm - vsp_ref[...], qsp_ref[...] - q_calc)
    if final:
        oa_ref[...] = f_p
        ob_ref[...] = f_q
    else:
        oa_ref[...] = va - _STEP * f_p
        ob_ref[...] = jnp.clip(vm - _STEP * f_q, _VM_MIN, _VM_MAX)


def _node_pass(final, pq, va2, vm2, psp, qsp, gs, bs, bt, vsp):
    p_parts, q_parts = pq
    rows = va2.shape[0]
    p3 = p_parts.reshape(_NSC, rows, 128)
    q3 = q_parts.reshape(_NSC, rows, 128)
    out_sd = jax.ShapeDtypeStruct((rows, 128), jnp.float32)
    return pl.pallas_call(
        functools.partial(_node_body, final),
        out_shape=(out_sd, out_sd),
    )(p3, q3, va2, vm2, psp, qsp, gs, bs, bt, vsp)


def kernel(x, edge_index, edge_attr, p_spec, q_spec, node_gs, node_bs,
           bus_type, vm_sp):
    n = x.shape[1] // 2
    e_total = edge_index.shape[1]
    n_pad = ((n + 2047) // 2048) * 2048
    rows = n_pad // 128
    ept = e_total // _NW
    chunk = 1000
    assert ept % chunk == 0 and (ept // chunk) % 4 == 0 and chunk % 8 == 0

    pad = n_pad - n
    vap = jnp.pad(x[0, :n], (0, pad))
    vmp = jnp.pad(x[0, n:], (0, pad))
    src = edge_index[0]
    dst = edge_index[1]

    e_rows = e_total // 128
    att = jnp.transpose(edge_attr.reshape(e_rows, 128, 8), (0, 2, 1))
    econ = _precompute_econ(att, e_rows, 200).reshape(8 * e_total)

    def p2(a):
        return jnp.pad(a[0], (0, pad)).reshape(rows, 128)

    psp = p2(p_spec)
    qsp = p2(q_spec)
    gs = p2(node_gs)
    bs = p2(node_bs)
    vsp = p2(vm_sp)
    bt = jnp.pad(bus_type[0], (0, pad), constant_values=1).reshape(rows, 128)

    va2 = vap.reshape(rows, 128)
    vm2 = vmp.reshape(rows, 128)
    for _ in range(_N_ITERS):
        pq = _edge_pass(va2.reshape(n_pad), vm2.reshape(n_pad), src, dst,
                        econ, n_pad, ept, chunk, e_total)
        va2, vm2 = _node_pass(False, pq, va2, vm2, psp, qsp, gs, bs, bt, vsp)
    pq = _edge_pass(va2.reshape(n_pad), vm2.reshape(n_pad), src, dst,
                    econ, n_pad, ept, chunk, e_total)
    f_p, f_q = _node_pass(True, pq, va2, vm2, psp, qsp, gs, bs, bt, vsp)
    return jnp.concatenate([f_p.reshape(n_pad)[:n][None, :],
                            f_q.reshape(n_pad)[:n][None, :]], axis=1)
